# Initial kernel scaffold; baseline (speedup 1.0000x reference)
import jax
import jax.numpy as jnp
from jax.experimental import pallas as pl


def kernel(x, table):
    raise NotImplementedError("write your pallas kernel here")


# speedup vs baseline: 1.6662x; 1.6662x over previous
import jax
import jax.numpy as jnp
from jax.experimental import pallas as pl

_BLOCK_P = 1024


def _add_kernel(x_ref, t_ref, o_ref):
    o_ref[0, :, :] = x_ref[0, :, :] + t_ref[:, :]


def kernel(x, table):
    batch, num_pos, feat = x.shape
    grid = (num_pos // _BLOCK_P, batch)
    return pl.pallas_call(
        _add_kernel,
        grid=grid,
        in_specs=[
            pl.BlockSpec((1, _BLOCK_P, feat), lambda ip, ib: (ib, ip, 0)),
            pl.BlockSpec((_BLOCK_P, feat), lambda ip, ib: (ip, 0)),
        ],
        out_specs=pl.BlockSpec((1, _BLOCK_P, feat), lambda ip, ib: (ib, ip, 0)),
        out_shape=jax.ShapeDtypeStruct(x.shape, x.dtype),
    )(x, table)


# speedup vs baseline: 1.7394x; 1.0440x over previous
import jax
import jax.numpy as jnp
from jax.experimental import pallas as pl

_BLOCK_P = 2048


def _add_kernel(x_ref, t_ref, o_ref):
    o_ref[0, :, :] = x_ref[0, :, :] + t_ref[:, :]


def kernel(x, table):
    batch, num_pos, feat = x.shape
    grid = (num_pos // _BLOCK_P, batch)
    return pl.pallas_call(
        _add_kernel,
        grid=grid,
        in_specs=[
            pl.BlockSpec((1, _BLOCK_P, feat), lambda ip, ib: (ib, ip, 0)),
            pl.BlockSpec((_BLOCK_P, feat), lambda ip, ib: (ip, 0)),
        ],
        out_specs=pl.BlockSpec((1, _BLOCK_P, feat), lambda ip, ib: (ib, ip, 0)),
        out_shape=jax.ShapeDtypeStruct(x.shape, x.dtype),
    )(x, table)
